# baseline (speedup 1.0000x reference)
import functools

import jax
import jax.numpy as jnp
from jax.experimental import pallas as pl
from jax.experimental.pallas import tpu as pltpu


def _mm_body(x_ref, w_ref, b_ref, o_ref, *, relu):
    acc = jnp.dot(x_ref[...], w_ref[...], preferred_element_type=jnp.float32)
    acc = acc + b_ref[...]
    if relu:
        acc = jnp.maximum(acc, 0.0)
    o_ref[...] = acc


def _mm_add_body(x_ref, w_ref, c_ref, o_ref, *, relu):
    acc = jnp.dot(x_ref[...], w_ref[...], preferred_element_type=jnp.float32)
    acc = acc + c_ref[...]
    if relu:
        acc = jnp.maximum(acc, 0.0)
    o_ref[...] = acc


def _mm(x, w, b=None, relu=False, block_m=1024):
    m, k = x.shape
    n = w.shape[1]
    if b is None:
        b = jnp.zeros((n,), jnp.float32)
    bm = min(block_m, m)
    grid = (pl.cdiv(m, bm),)
    return pl.pallas_call(
        functools.partial(_mm_body, relu=relu),
        grid=grid,
        in_specs=[
            pl.BlockSpec((bm, k), lambda i: (i, 0)),
            pl.BlockSpec((k, n), lambda i: (0, 0)),
            pl.BlockSpec((1, n), lambda i: (0, 0)),
        ],
        out_specs=pl.BlockSpec((bm, n), lambda i: (i, 0)),
        out_shape=jax.ShapeDtypeStruct((m, n), jnp.float32),
    )(x, w, b[None, :])


def _mm_add(x, w, c, relu=False, block_m=1024):
    m, k = x.shape
    n = w.shape[1]
    bm = min(block_m, m)
    grid = (pl.cdiv(m, bm),)
    return pl.pallas_call(
        functools.partial(_mm_add_body, relu=relu),
        grid=grid,
        in_specs=[
            pl.BlockSpec((bm, k), lambda i: (i, 0)),
            pl.BlockSpec((k, n), lambda i: (0, 0)),
            pl.BlockSpec((bm, n), lambda i: (i, 0)),
        ],
        out_specs=pl.BlockSpec((bm, n), lambda i: (i, 0)),
        out_shape=jax.ShapeDtypeStruct((m, n), jnp.float32),
    )(x, w, c)


def _set2set(h_act, gid, s2s_Wih0, s2s_Whh0, s2s_b0, s2s_Wih1, s2s_Whh1,
             s2s_b1):
    n, d = h_act.shape
    g = 64
    mask = gid[:, None] == jnp.arange(g)[None, :]

    def lstm(x, h, c, Wih, Whh, b):
        z = x @ Wih + h @ Whh + b
        i, f, gg, o = jnp.split(z, 4, axis=-1)
        c = jax.nn.sigmoid(f) * c + jax.nn.sigmoid(i) * jnp.tanh(gg)
        h = jax.nn.sigmoid(o) * jnp.tanh(c)
        return h, c

    q_star = jnp.zeros((g, 2 * d), jnp.float32)
    h1 = jnp.zeros((g, d), jnp.float32)
    c1 = jnp.zeros((g, d), jnp.float32)
    h2 = jnp.zeros((g, d), jnp.float32)
    c2 = jnp.zeros((g, d), jnp.float32)
    neg_inf = jnp.float32(-jnp.inf)
    for _ in range(6):
        h1, c1 = lstm(q_star, h1, c1, s2s_Wih0, s2s_Whh0, s2s_b0)
        h2, c2 = lstm(h1, h2, c2, s2s_Wih1, s2s_Whh1, s2s_b1)
        q = h2
        att = h_act @ q.T
        attm = jnp.where(mask, att, neg_inf)
        mmax = jnp.max(attm, axis=0)
        a_un = jnp.where(mask, jnp.exp(attm - mmax[None, :]), 0.0)
        s = jnp.sum(a_un, axis=0)
        a = a_un / (s + 1e-9)[None, :]
        r = a.T @ h_act
        q_star = jnp.concatenate([q, r], axis=1)
    return q_star


def kernel(x_node, x_edge, edge_index, node_graph_ids, enc_W_node, enc_b_node,
           enc_W_edge, enc_b_edge, enc_W_msg, enc_W_self, enc_b_msg,
           edge_mlp_W1, edge_mlp_b1, edge_mlp_W2, edge_mlp_b2, s2s_Wih0,
           s2s_Whh0, s2s_b0, s2s_Wih1, s2s_Whh1, s2s_b1, cls_act_W1,
           cls_act_b1, cls_act_W2, cls_act_b2, cls_del_W1, cls_del_b1,
           cls_del_W2, cls_del_b2, cls_add_W1, cls_add_b1, cls_add_W2,
           cls_add_b2, cls_arm_W1, cls_arm_b1, cls_arm_W2, cls_arm_b2):
    n, d = x_node.shape
    e_cnt, de = x_edge.shape
    n_layers = enc_W_msg.shape[1]
    src = edge_index[0]
    dst = edge_index[1]

    def encode(e):
        h = _mm(x_node, enc_W_node[e], enc_b_node[e], relu=True)
        he = _mm(x_edge, enc_W_edge[e], enc_b_edge[e], relu=True)
        for l in range(n_layers):
            w_msg = enc_W_msg[e, l]
            hW = _mm(h, w_msg[:d])
            heW = _mm(he, w_msg[d:], b=enc_b_msg[e, l])
            m = jnp.maximum(hW[src] + heW, 0.0)
            agg = jax.ops.segment_sum(m, dst, num_segments=n)
            h = _mm_add(h, enc_W_self[e, l], agg, relu=True)
        return h

    h_act = encode(0)
    h_del = encode(1)
    h_add = encode(2)
    h_arm = encode(3)

    q_star = _set2set(h_act, node_graph_ids, s2s_Wih0, s2s_Whh0, s2s_b0,
                      s2s_Wih1, s2s_Whh1, s2s_b1)
    pred_act = jax.nn.relu(q_star @ cls_act_W1 + cls_act_b1) @ cls_act_W2 \
        + cls_act_b2

    h_e = _mm(_mm(x_edge, edge_mlp_W1, edge_mlp_b1, relu=True),
              edge_mlp_W2, edge_mlp_b2)
    zA = _mm(h_del, cls_del_W1[:d])
    zB = _mm(h_del, cls_del_W1[d + de:])
    zE = _mm(h_e, cls_del_W1[d:d + de], b=cls_del_b1)
    z = jnp.maximum(zA[src] + zE + zB[dst], 0.0)
    pred_del = z @ cls_del_W2 + cls_del_b2

    pred_add = _mm(h_add, cls_add_W1, cls_add_b1, relu=True) @ cls_add_W2 \
        + cls_add_b2
    pred_arm = _mm(h_arm, cls_arm_W1, cls_arm_b1, relu=True) @ cls_arm_W2 \
        + cls_arm_b2
    return (pred_act, pred_del, pred_add, pred_arm)
